# baseline (speedup 1.0000x reference)
import functools

import jax
import jax.numpy as jnp
from jax import lax
from jax.experimental import pallas as pl
from jax.experimental.pallas import tpu as pltpu
from jax.experimental.pallas import tpu_sc as plsc

EMBED = 4096
SEQLEN = 8192
BATCH = 4

NC = 2
NS = 16
NW = NC * NS
LANES = 16

ROWS_TOTAL = BATCH * SEQLEN
ROWS_PER_W = ROWS_TOTAL // NW
TILE_R = 8
NTILES = ROWS_PER_W // TILE_R
KSTEPS = EMBED // LANES


def _sc_compute_tile(buf, w_v, row0, bv, bi):
    accs = tuple(jnp.zeros((LANES,), jnp.float32) for _ in range(TILE_R))

    def kbody(k, accs):
        off = pl.multiple_of(k * LANES, LANES)
        wv = w_v[pl.ds(off, LANES)]
        return tuple(accs[r] + buf[r, pl.ds(off, LANES)] * wv
                     for r in range(TILE_R))

    accs = lax.fori_loop(0, KSTEPS, kbody, accs, unroll=2)
    for r in range(TILE_R):
        s = jnp.sum(accs[r])
        upd = s > bv
        bi = jnp.where(upd, row0 + r, bi)
        bv = jnp.where(upd, s, bv)
    return bv, bi


def _sc_score_body(x_hbm, w_hbm, out_val, out_idx,
                   w_v, buf0, buf1, vbuf, ibuf, sem0, sem1):
    wid = lax.axis_index("s") * NC + lax.axis_index("c")
    base = wid * ROWS_PER_W

    pltpu.sync_copy(w_hbm.at[0], w_v)

    def issue(t, buf, sem):
        pltpu.async_copy(x_hbm.at[pl.ds(base + t * TILE_R, TILE_R), :], buf, sem)

    def wait(buf, sem):
        pltpu.make_async_copy(x_hbm.at[pl.ds(base, TILE_R), :], buf, sem).wait()

    issue(0, buf0, sem0)
    issue(1, buf1, sem1)

    def body2(t2, carry):
        bv, bi = carry
        t0 = t2 * 2
        wait(buf0, sem0)
        bv, bi = _sc_compute_tile(buf0, w_v, base + t0 * TILE_R, bv, bi)

        @pl.when(t0 + 2 < NTILES)
        def _():
            issue(t0 + 2, buf0, sem0)

        wait(buf1, sem1)
        bv, bi = _sc_compute_tile(buf1, w_v, base + (t0 + 1) * TILE_R, bv, bi)

        @pl.when(t0 + 3 < NTILES)
        def _():
            issue(t0 + 3, buf1, sem1)

        return bv, bi

    bv, bi = lax.fori_loop(0, NTILES // 2, body2,
                           (jnp.float32(-jnp.inf), jnp.int32(0)))

    vbuf[...] = jnp.broadcast_to(bv, (LANES,))
    ibuf[...] = jnp.broadcast_to(bi, (LANES,))
    pltpu.sync_copy(vbuf, out_val.at[wid])
    pltpu.sync_copy(ibuf, out_idx.at[wid])


def _sc_score(x_flat, W):
    mesh = plsc.VectorSubcoreMesh(core_axis_name="c", subcore_axis_name="s")
    return pl.kernel(
        _sc_score_body,
        out_type=(
            jax.ShapeDtypeStruct((NW, LANES), jnp.float32),
            jax.ShapeDtypeStruct((NW, LANES), jnp.int32),
        ),
        mesh=mesh,
        scratch_types=[
            pltpu.VMEM((EMBED,), jnp.float32),
            pltpu.VMEM((TILE_R, EMBED), jnp.float32),
            pltpu.VMEM((TILE_R, EMBED), jnp.float32),
            pltpu.VMEM((LANES,), jnp.float32),
            pltpu.VMEM((LANES,), jnp.int32),
            pltpu.SemaphoreType.DMA,
            pltpu.SemaphoreType.DMA,
        ],
        compiler_params=pltpu.CompilerParams(needs_layout_passes=False),
    )(x_flat, W)


def _merge_body(vals_ref, idx_ref, x_ref, emb_ref, oidx_ref, row_v, sem):
    b = pl.program_id(0)
    vblock = vals_ref[pl.ds(b * (NW // BATCH), NW // BATCH), :]
    iblock = idx_ref[pl.ds(b * (NW // BATCH), NW // BATCH), :]
    m = jnp.max(vblock)
    gi = jnp.min(jnp.where(vblock == m, iblock, jnp.int32(2**30)))
    oidx_ref[b] = gi - b * SEQLEN

    cp = pltpu.make_async_copy(x_ref.at[pl.ds(gi, 1), :], row_v, sem)
    cp.start()
    cp.wait()
    emb_ref[0] = row_v[...]


def _merge_gather(x_flat, vals, idxs):
    emb, oidx = pl.pallas_call(
        _merge_body,
        grid=(BATCH,),
        in_specs=[
            pl.BlockSpec((NW, LANES), lambda b: (0, 0)),
            pl.BlockSpec((NW, LANES), lambda b: (0, 0)),
            pl.BlockSpec(memory_space=pltpu.HBM),
        ],
        out_specs=[
            pl.BlockSpec((1, 1, EMBED), lambda b: (b, 0, 0)),
            pl.BlockSpec(memory_space=pltpu.SMEM),
        ],
        out_shape=[
            jax.ShapeDtypeStruct((BATCH, 1, EMBED), jnp.float32),
            jax.ShapeDtypeStruct((BATCH,), jnp.int32),
        ],
        scratch_shapes=[
            pltpu.VMEM((1, EMBED), jnp.float32),
            pltpu.SemaphoreType.DMA,
        ],
    )(vals, idxs, x_flat)
    return emb.reshape(BATCH, EMBED), oidx


@jax.jit
def _run(token_embeddings, W):
    x_flat = token_embeddings.reshape(ROWS_TOTAL, EMBED)
    vals, idxs = _sc_score(x_flat, W)
    return _merge_gather(x_flat, vals, idxs)


def kernel(token_embeddings, W, b):
    emb, idx = _run(token_embeddings, W)
    return emb, idx


# speedup vs baseline: 1.3091x; 1.3091x over previous
import functools

import jax
import jax.numpy as jnp
from jax import lax
from jax.experimental import pallas as pl
from jax.experimental.pallas import tpu as pltpu
from jax.experimental.pallas import tpu_sc as plsc

EMBED = 4096
SEQLEN = 8192
BATCH = 4

S_TC = 5120
S_SC = SEQLEN - S_TC

NC = 2
NS = 16
NW = NC * NS
LANES = 16
W_PER_B = NW // BATCH
ROWS_PER_W = S_SC // W_PER_B
TILE_R = 8
NTILES = ROWS_PER_W // TILE_R
KSTEPS = EMBED // LANES

BLOCK_S = 512
NBLOCKS_TC = S_TC // BLOCK_S


def _sc_compute_tile(buf, w_v, row0, bv, bi):
    accs = tuple(jnp.zeros((LANES,), jnp.float32) for _ in range(TILE_R))

    def kbody(k, accs):
        off = pl.multiple_of(k * LANES, LANES)
        wv = w_v[pl.ds(off, LANES)]
        return tuple(accs[r] + buf[r, pl.ds(off, LANES)] * wv
                     for r in range(TILE_R))

    accs = lax.fori_loop(0, KSTEPS, kbody, accs, unroll=2)
    for r in range(TILE_R):
        s = jnp.sum(accs[r])
        upd = s > bv
        bi = jnp.where(upd, row0 + r, bi)
        bv = jnp.where(upd, s, bv)
    return bv, bi


def _sc_score_body(x_hbm, w_hbm, out_val, out_idx,
                   w_v, buf0, buf1, vbuf, ibuf, sem0, sem1):
    wid = lax.axis_index("s") * NC + lax.axis_index("c")
    b = wid // W_PER_B
    j = wid % W_PER_B
    base = b * SEQLEN + S_TC + j * ROWS_PER_W

    pltpu.sync_copy(w_hbm.at[0], w_v)

    def issue(t, buf, sem):
        pltpu.async_copy(x_hbm.at[pl.ds(base + t * TILE_R, TILE_R), :], buf, sem)

    def wait(buf, sem):
        pltpu.make_async_copy(x_hbm.at[pl.ds(base, TILE_R), :], buf, sem).wait()

    issue(0, buf0, sem0)
    issue(1, buf1, sem1)

    def body2(t2, carry):
        bv, bi = carry
        t0 = t2 * 2
        wait(buf0, sem0)
        bv, bi = _sc_compute_tile(buf0, w_v, base + t0 * TILE_R, bv, bi)

        @pl.when(t0 + 2 < NTILES)
        def _():
            issue(t0 + 2, buf0, sem0)

        wait(buf1, sem1)
        bv, bi = _sc_compute_tile(buf1, w_v, base + (t0 + 1) * TILE_R, bv, bi)

        @pl.when(t0 + 3 < NTILES)
        def _():
            issue(t0 + 3, buf1, sem1)

        return bv, bi

    bv, bi = lax.fori_loop(0, NTILES // 2, body2,
                           (jnp.float32(-jnp.inf), jnp.int32(0)))

    vbuf[...] = jnp.broadcast_to(bv, (LANES,))
    ibuf[...] = jnp.broadcast_to(bi, (LANES,))
    pltpu.sync_copy(vbuf, out_val.at[wid])
    pltpu.sync_copy(ibuf, out_idx.at[wid])


def _sc_score(x_flat, W):
    mesh = plsc.VectorSubcoreMesh(core_axis_name="c", subcore_axis_name="s")
    return pl.kernel(
        _sc_score_body,
        out_type=(
            jax.ShapeDtypeStruct((NW, LANES), jnp.float32),
            jax.ShapeDtypeStruct((NW, LANES), jnp.int32),
        ),
        mesh=mesh,
        scratch_types=[
            pltpu.VMEM((EMBED,), jnp.float32),
            pltpu.VMEM((TILE_R, EMBED), jnp.float32),
            pltpu.VMEM((TILE_R, EMBED), jnp.float32),
            pltpu.VMEM((LANES,), jnp.float32),
            pltpu.VMEM((LANES,), jnp.int32),
            pltpu.SemaphoreType.DMA,
            pltpu.SemaphoreType.DMA,
        ],
        compiler_params=pltpu.CompilerParams(needs_layout_passes=False),
    )(x_flat, W)


def _tc_score_body(x_ref, w_ref, vals_ref, idx_ref, mval_ref):
    b = pl.program_id(0)
    s = pl.program_id(1)

    @pl.when(s == 0)
    def _init():
        mval_ref[0] = -jnp.inf

    x = x_ref[0]
    w = w_ref[...]
    logits = jnp.sum(x * w, axis=1, keepdims=True)

    m = jnp.max(logits)
    row_ids = jax.lax.broadcasted_iota(jnp.int32, logits.shape, 0)
    local_idx = jnp.min(jnp.where(logits == m, row_ids, BLOCK_S))

    @pl.when(m > mval_ref[0])
    def _update():
        mval_ref[0] = m
        vals_ref[b] = m
        idx_ref[b] = s * BLOCK_S + local_idx


def _tc_score(token_embeddings, W):
    return pl.pallas_call(
        _tc_score_body,
        grid=(BATCH, NBLOCKS_TC),
        in_specs=[
            pl.BlockSpec((1, BLOCK_S, EMBED), lambda b, s: (b, s, 0)),
            pl.BlockSpec((1, EMBED), lambda b, s: (0, 0)),
        ],
        out_specs=[
            pl.BlockSpec(memory_space=pltpu.SMEM),
            pl.BlockSpec(memory_space=pltpu.SMEM),
        ],
        out_shape=[
            jax.ShapeDtypeStruct((BATCH,), jnp.float32),
            jax.ShapeDtypeStruct((BATCH,), jnp.int32),
        ],
        scratch_shapes=[pltpu.SMEM((1,), jnp.float32)],
        compiler_params=pltpu.CompilerParams(
            dimension_semantics=("arbitrary", "arbitrary"),
        ),
    )(token_embeddings, W)


def _merge_body(svals_ref, sidx_ref, tvals_ref, tidx_ref, x_ref,
                emb_ref, oidx_ref, row_v, sem):
    b = pl.program_id(0)
    vblock = svals_ref[pl.ds(b * W_PER_B, W_PER_B), :]
    iblock = sidx_ref[pl.ds(b * W_PER_B, W_PER_B), :]
    sc_m = jnp.max(vblock)
    sc_i = jnp.min(jnp.where(vblock == sc_m, iblock, jnp.int32(2**30)))

    tc_v = tvals_ref[b]
    tc_i = b * SEQLEN + tidx_ref[b]

    use_tc = tc_v >= sc_m
    gi = jnp.where(use_tc, tc_i, sc_i)
    oidx_ref[b] = gi - b * SEQLEN

    cp = pltpu.make_async_copy(x_ref.at[pl.ds(gi, 1), :], row_v, sem)
    cp.start()
    cp.wait()
    emb_ref[0] = row_v[...]


def _merge_gather(x_flat, svals, sidx, tvals, tidx):
    emb, oidx = pl.pallas_call(
        _merge_body,
        grid=(BATCH,),
        in_specs=[
            pl.BlockSpec((NW, LANES), lambda b: (0, 0)),
            pl.BlockSpec((NW, LANES), lambda b: (0, 0)),
            pl.BlockSpec(memory_space=pltpu.SMEM),
            pl.BlockSpec(memory_space=pltpu.SMEM),
            pl.BlockSpec(memory_space=pltpu.HBM),
        ],
        out_specs=[
            pl.BlockSpec((1, 1, EMBED), lambda b: (b, 0, 0)),
            pl.BlockSpec(memory_space=pltpu.SMEM),
        ],
        out_shape=[
            jax.ShapeDtypeStruct((BATCH, 1, EMBED), jnp.float32),
            jax.ShapeDtypeStruct((BATCH,), jnp.int32),
        ],
        scratch_shapes=[
            pltpu.VMEM((1, EMBED), jnp.float32),
            pltpu.SemaphoreType.DMA,
        ],
    )(svals, sidx, tvals, tidx, x_flat)
    return emb.reshape(BATCH, EMBED), oidx


@jax.jit
def _run(token_embeddings, W):
    x_flat = token_embeddings.reshape(BATCH * SEQLEN, EMBED)
    svals, sidx = _sc_score(x_flat, W)
    tvals, tidx = _tc_score(token_embeddings, W)
    return _merge_gather(x_flat, svals, sidx, tvals, tidx)


def kernel(token_embeddings, W, b):
    emb, idx = _run(token_embeddings, W)
    return emb, idx


# speedup vs baseline: 1.5554x; 1.1881x over previous
import functools

import jax
import jax.numpy as jnp
from jax.experimental import pallas as pl
from jax.experimental.pallas import tpu as pltpu

EMBED = 4096
SEQLEN = 8192
BLOCK_S = 1024


def _selector_body(x_ref, w_ref, emb_ref, idx_ref, mval_ref):
    b = pl.program_id(0)
    s = pl.program_id(1)

    @pl.when(s == 0)
    def _init():
        mval_ref[0] = -jnp.inf

    x = x_ref[0]
    w = w_ref[...]
    logits = jnp.sum(x * w, axis=1, keepdims=True)

    m = jnp.max(logits)
    row_ids = jax.lax.broadcasted_iota(jnp.int32, logits.shape, 0)
    local_idx = jnp.min(jnp.where(logits == m, row_ids, BLOCK_S))

    @pl.when(m > mval_ref[0])
    def _update():
        mval_ref[0] = m
        idx_ref[b] = s * BLOCK_S + local_idx
        emb_ref[0] = x_ref[0, pl.ds(local_idx, 1), :]


@jax.jit
def _run(token_embeddings, W):
    B = token_embeddings.shape[0]
    grid = (B, SEQLEN // BLOCK_S)
    emb, idx = pl.pallas_call(
        _selector_body,
        grid=grid,
        in_specs=[
            pl.BlockSpec((1, BLOCK_S, EMBED), lambda b, s: (b, s, 0)),
            pl.BlockSpec((1, EMBED), lambda b, s: (0, 0)),
        ],
        out_specs=[
            pl.BlockSpec((1, 1, EMBED), lambda b, s: (b, 0, 0)),
            pl.BlockSpec(memory_space=pltpu.SMEM),
        ],
        out_shape=[
            jax.ShapeDtypeStruct((B, 1, EMBED), jnp.float32),
            jax.ShapeDtypeStruct((B,), jnp.int32),
        ],
        scratch_shapes=[pltpu.SMEM((1,), jnp.float32)],
        compiler_params=pltpu.CompilerParams(
            dimension_semantics=("arbitrary", "arbitrary"),
        ),
    )(token_embeddings, W)
    return emb.reshape(B, EMBED), idx


def kernel(token_embeddings, W, b):
    emb, idx = _run(token_embeddings, W)
    return emb, idx


# speedup vs baseline: 1.5660x; 1.0068x over previous
import functools

import jax
import jax.numpy as jnp
from jax.experimental import pallas as pl
from jax.experimental.pallas import tpu as pltpu

EMBED = 4096
SEQLEN = 8192
BLOCK_S = 1024


def _selector_body(x_ref, w_ref, emb_ref, idx_ref, mval_ref):
    b = pl.program_id(0)
    s = pl.program_id(1)

    @pl.when(s == 0)
    def _init():
        mval_ref[0] = -jnp.inf

    x = x_ref[0]
    w = w_ref[...]
    logits = jnp.sum(x * w, axis=1, keepdims=True)

    m = jnp.max(logits)
    row_ids = jax.lax.broadcasted_iota(jnp.int32, logits.shape, 0)
    local_idx = jnp.min(jnp.where(logits == m, row_ids, BLOCK_S))

    @pl.when(m > mval_ref[0])
    def _update():
        mval_ref[0] = m
        idx_ref[b] = s * BLOCK_S + local_idx
        emb_ref[pl.ds(b, 1), :] = x_ref[0, pl.ds(local_idx, 1), :]


@jax.jit
def _run(token_embeddings, W):
    B = token_embeddings.shape[0]
    grid = (B, SEQLEN // BLOCK_S)
    emb, idx = pl.pallas_call(
        _selector_body,
        grid=grid,
        in_specs=[
            pl.BlockSpec((1, BLOCK_S, EMBED), lambda b, s: (b, s, 0)),
            pl.BlockSpec((1, EMBED), lambda b, s: (0, 0)),
        ],
        out_specs=[
            pl.BlockSpec((B, EMBED), lambda b, s: (0, 0)),
            pl.BlockSpec(memory_space=pltpu.SMEM),
        ],
        out_shape=[
            jax.ShapeDtypeStruct((B, EMBED), jnp.float32),
            jax.ShapeDtypeStruct((B,), jnp.int32),
        ],
        scratch_shapes=[pltpu.SMEM((1,), jnp.float32)],
        compiler_params=pltpu.CompilerParams(
            dimension_semantics=("arbitrary", "arbitrary"),
        ),
    )(token_embeddings, W)
    return emb, idx


def kernel(token_embeddings, W, b):
    emb, idx = _run(token_embeddings, W)
    return emb, idx


# speedup vs baseline: 1.5717x; 1.0037x over previous
import functools

import jax
import jax.numpy as jnp
from jax.experimental import pallas as pl
from jax.experimental.pallas import tpu as pltpu

EMBED = 4096
HALF = EMBED // 2
SEQLEN = 8192
BLOCK_S = 1024


def _selector_body(xa_ref, xb_ref, wa_ref, wb_ref, emb_ref, idx_ref, mval_ref):
    b = pl.program_id(0)
    s = pl.program_id(1)

    @pl.when(s == 0)
    def _init():
        mval_ref[0] = -jnp.inf

    xa = xa_ref[0]
    xb = xb_ref[0]
    logits = (jnp.sum(xa * wa_ref[...], axis=1, keepdims=True)
              + jnp.sum(xb * wb_ref[...], axis=1, keepdims=True))

    m = jnp.max(logits)
    row_ids = jax.lax.broadcasted_iota(jnp.int32, logits.shape, 0)
    local_idx = jnp.min(jnp.where(logits == m, row_ids, BLOCK_S))

    @pl.when(m > mval_ref[0])
    def _update():
        mval_ref[0] = m
        idx_ref[b] = s * BLOCK_S + local_idx
        emb_ref[pl.ds(b, 1), 0:HALF] = xa_ref[0, pl.ds(local_idx, 1), :]
        emb_ref[pl.ds(b, 1), HALF:EMBED] = xb_ref[0, pl.ds(local_idx, 1), :]


@jax.jit
def _run(token_embeddings, W):
    B = token_embeddings.shape[0]
    grid = (B, SEQLEN // BLOCK_S)
    emb, idx = pl.pallas_call(
        _selector_body,
        grid=grid,
        in_specs=[
            pl.BlockSpec((1, BLOCK_S, HALF), lambda b, s: (b, s, 0)),
            pl.BlockSpec((1, BLOCK_S, HALF), lambda b, s: (b, s, 1)),
            pl.BlockSpec((1, HALF), lambda b, s: (0, 0)),
            pl.BlockSpec((1, HALF), lambda b, s: (0, 1)),
        ],
        out_specs=[
            pl.BlockSpec((B, EMBED), lambda b, s: (0, 0)),
            pl.BlockSpec(memory_space=pltpu.SMEM),
        ],
        out_shape=[
            jax.ShapeDtypeStruct((B, EMBED), jnp.float32),
            jax.ShapeDtypeStruct((B,), jnp.int32),
        ],
        scratch_shapes=[pltpu.SMEM((1,), jnp.float32)],
        compiler_params=pltpu.CompilerParams(
            dimension_semantics=("arbitrary", "arbitrary"),
        ),
    )(token_embeddings, token_embeddings, W, W)
    return emb, idx


def kernel(token_embeddings, W, b):
    emb, idx = _run(token_embeddings, W)
    return emb, idx
